# baseline (speedup 1.0000x reference)
import functools

import jax
import jax.numpy as jnp
from jax import lax
from jax.experimental import pallas as pl
from jax.experimental.pallas import tpu as pltpu
from jax.experimental.pallas import tpu_sc as plsc

N = 10000
E = 320000
NPAD = 10016
NC, NS = 2, 16
NW = NC * NS
EPW = E // NW
CH = 112
NCH = -(-EPW // CH)
EPW_PAD = NCH * CH
RPT = NPAD // NS
DB = 16

_mesh = plsc.VectorSubcoreMesh(core_axis_name="c", subcore_axis_name="s",
                               num_cores=NC, num_subcores=NS)


def _make_sc_scatter(F):
    @functools.partial(
        pl.kernel,
        out_type=jax.ShapeDtypeStruct((NC, NPAD, F), jnp.float32),
        mesh=_mesh,
        compiler_params=pltpu.CompilerParams(use_tc_tiling_on_sc=False),
        scratch_types=[
            pltpu.VMEM((NCH, CH), jnp.int32),
            pltpu.VMEM((NCH, CH), jnp.int32),
            pltpu.VMEM((CH, F), jnp.float32),
            pltpu.VMEM((CH, F), jnp.float32),
            pltpu.VMEM_SHARED((NPAD, F), jnp.float32),
            pltpu.SemaphoreType.DMA,
            pltpu.SemaphoreType.DMA,
            pltpu.SemaphoreType.DMA,
            pltpu.SemaphoreType.DMA,
        ],
    )
    def k(g_hbm, src_hbm, dst_hbm, zeros_hbm, out_hbm,
          srciv, dstiv, gbuf0, gbuf1, acc, semg0, semg1, sems0, sems1):
        c = lax.axis_index("c")
        s = lax.axis_index("s")
        w = c * NS + s
        pltpu.sync_copy(src_hbm.at[w], srciv)
        pltpu.sync_copy(dst_hbm.at[w], dstiv)
        base = s * RPT
        pltpu.sync_copy(zeros_hbm, acc.at[pl.ds(base, RPT)])
        plsc.subcore_barrier()

        def gath_start(j, buf, sem):
            pltpu.async_copy(g_hbm.at[srciv.at[j]], buf, sem)

        def gath_wait(j, buf, sem):
            pltpu.make_async_copy(g_hbm.at[srciv.at[j]], buf, sem).wait()

        def scat_start(j, buf, sem):
            pltpu.async_copy(buf, acc.at[dstiv.at[j]], sem, add=True)

        def scat_wait(j, buf, sem):
            pltpu.make_async_copy(buf, acc.at[dstiv.at[j]], sem).wait()

        gath_start(0, gbuf0, semg0)
        gath_wait(0, gbuf0, semg0)
        scat_start(0, gbuf0, sems0)
        gath_start(1, gbuf1, semg1)
        gath_wait(1, gbuf1, semg1)
        scat_start(1, gbuf1, sems1)
        scat_wait(0, gbuf0, sems0)
        gath_start(2, gbuf0, semg0)

        def body(jj, carry):
            j = jj * 2 + 2
            gath_wait(j, gbuf0, semg0)
            scat_start(j, gbuf0, sems0)
            scat_wait(j - 1, gbuf1, sems1)
            gath_start(j + 1, gbuf1, semg1)
            gath_wait(j + 1, gbuf1, semg1)
            scat_start(j + 1, gbuf1, sems1)
            scat_wait(j, gbuf0, sems0)
            gath_start(j + 2, gbuf0, semg0)
            return carry

        lax.fori_loop(0, (NCH - 4) // 2, body, 0)
        gath_wait(NCH - 2, gbuf0, semg0)
        scat_start(NCH - 2, gbuf0, sems0)
        scat_wait(NCH - 3, gbuf1, sems1)
        gath_start(NCH - 1, gbuf1, semg1)
        gath_wait(NCH - 1, gbuf1, semg1)
        scat_start(NCH - 1, gbuf1, sems1)
        scat_wait(NCH - 2, gbuf0, sems0)
        scat_wait(NCH - 1, gbuf1, sems1)
        plsc.subcore_barrier()
        pltpu.sync_copy(acc.at[pl.ds(base, RPT)], out_hbm.at[c].at[pl.ds(base, RPT)])

    return k


@functools.partial(
    pl.kernel,
    out_type=jax.ShapeDtypeStruct((NC, NPAD, DB), jnp.float32),
    mesh=_mesh,
    compiler_params=pltpu.CompilerParams(use_tc_tiling_on_sc=False),
    scratch_types=[
        pltpu.VMEM((NCH, CH), jnp.int32),
        pltpu.VMEM((CH, DB), jnp.float32),
        pltpu.VMEM_SHARED((NPAD, DB), jnp.float32),
        pltpu.SemaphoreType.DMA,
    ],
)
def _sc_degree(dst_hbm, ones_hbm, zeros_hbm, out_hbm, dstiv, obuf, acc, sem):
    c = lax.axis_index("c")
    s = lax.axis_index("s")
    w = c * NS + s
    pltpu.sync_copy(dst_hbm.at[w], dstiv)
    pltpu.sync_copy(ones_hbm, obuf)
    base = s * RPT
    pltpu.sync_copy(zeros_hbm, acc.at[pl.ds(base, RPT)])
    plsc.subcore_barrier()

    def fire(j, carry):
        pltpu.async_copy(obuf, acc.at[dstiv.at[j]], sem, add=True)
        return carry

    def drain(j, carry):
        pltpu.make_async_copy(obuf, acc.at[dstiv.at[j]], sem).wait()
        return carry

    lax.fori_loop(0, NCH, fire, 0)
    lax.fori_loop(0, NCH, drain, 0)
    plsc.subcore_barrier()
    pltpu.sync_copy(acc.at[pl.ds(base, RPT)], out_hbm.at[c].at[pl.ds(base, RPT)])


_sc_scat128 = _make_sc_scatter(128)
_sc_scat64 = _make_sc_scatter(64)
_sc_scat32 = _make_sc_scatter(32)


def _tc1_body(d0, d1, x, w, dinv_o, p_o, g_o):
    deg = d0[:, 0:1] + d1[:, 0:1] + 1.0
    dinv = lax.rsqrt(deg)
    p = jnp.dot(x[...], w[...], preferred_element_type=jnp.float32, precision=lax.Precision.HIGHEST)
    dinv_o[...] = dinv
    p_o[...] = p
    g_o[...] = p * dinv


def _tc_stage1(d0, d1, x, w):
    f = w.shape[1]
    return pl.pallas_call(
        _tc1_body,
        out_shape=(
            jax.ShapeDtypeStruct((NPAD, 1), jnp.float32),
            jax.ShapeDtypeStruct((NPAD, f), jnp.float32),
            jax.ShapeDtypeStruct((NPAD, f), jnp.float32),
        ),
    )(d0, d1, x, w)


def _tc2_body(q0, q1, p_prev, dinv, b, w, p_o, g_o):
    dv = dinv[...]
    h = jnp.maximum((q0[...] + q1[...]) * dv + p_prev[...] * (dv * dv) + b[...], 0.0)
    p = jnp.dot(h, w[...], preferred_element_type=jnp.float32, precision=lax.Precision.HIGHEST)
    p_o[...] = p
    g_o[...] = p * dv


def _tc_stage2(q0, q1, p_prev, dinv, b, w):
    f = w.shape[1]
    return pl.pallas_call(
        _tc2_body,
        out_shape=(
            jax.ShapeDtypeStruct((NPAD, f), jnp.float32),
            jax.ShapeDtypeStruct((NPAD, f), jnp.float32),
        ),
    )(q0, q1, p_prev, dinv, b, w)


def _tc_final_body(q0, q1, p_prev, dinv, b, watt, fcw, fcb, sw, sb, out):
    dv = dinv[...]
    h3 = (q0[...] + q1[...]) * dv + p_prev[...] * (dv * dv) + b[...]
    mask = lax.broadcasted_iota(jnp.int32, (NPAD, 1), 0) < N
    h3m = jnp.where(mask, h3, 0.0)
    gw = jnp.dot(h3m, watt[...], preferred_element_type=jnp.float32, precision=lax.Precision.HIGHEST)
    gc = jnp.sum(gw, axis=0, keepdims=True) * (1.0 / N)
    tg = jnp.tanh(gc)
    ss = jax.nn.sigmoid(jnp.sum(h3m * tg, axis=1, keepdims=True))
    rep = jnp.sum(h3m * ss, axis=0, keepdims=True)
    s1 = jnp.maximum(jnp.dot(rep, fcw[...], preferred_element_type=jnp.float32, precision=lax.Precision.HIGHEST)
                     + fcb[...], 0.0)
    out[...] = jax.nn.sigmoid(
        jnp.dot(s1, sw[...], preferred_element_type=jnp.float32, precision=lax.Precision.HIGHEST) + sb[...])


def _tc_final(q0, q1, p_prev, dinv, b, watt, fcw, fcb, sw, sb):
    return pl.pallas_call(
        _tc_final_body,
        out_shape=jax.ShapeDtypeStruct((1, 1), jnp.float32),
    )(q0, q1, p_prev, dinv, b, watt, fcw, fcb, sw, sb)


def kernel(features_1, edge_index_1, W1, b1, W2, b2, W3, b3, Watt, fcW, fcb, sW, sb):
    ei = edge_index_1.astype(jnp.int32)
    src = ei[0].reshape(NW, EPW)
    dst = ei[1].reshape(NW, EPW)
    srcp = jnp.pad(src, ((0, 0), (0, EPW_PAD - EPW))).reshape(NW, NCH, CH)
    dstp = jnp.pad(dst, ((0, 0), (0, EPW_PAD - EPW)),
                   constant_values=N).reshape(NW, NCH, CH)
    xp = jnp.pad(features_1, ((0, NPAD - N), (0, 0)))

    z16 = jnp.zeros((RPT, DB), jnp.float32)
    z128 = jnp.zeros((RPT, 128), jnp.float32)
    z64 = jnp.zeros((RPT, 64), jnp.float32)
    z32 = jnp.zeros((RPT, 32), jnp.float32)
    ones16 = jnp.ones((CH, DB), jnp.float32)

    dpart = _sc_degree(dstp, ones16, z16)
    dinv, p1, g1 = _tc_stage1(dpart[0], dpart[1], xp, W1)

    part1 = _sc_scat128(g1, srcp, dstp, z128)
    p2, g2 = _tc_stage2(part1[0], part1[1], p1, dinv, b1.reshape(1, -1), W2)

    part2 = _sc_scat64(g2, srcp, dstp, z64)
    p3, g3 = _tc_stage2(part2[0], part2[1], p2, dinv, b2.reshape(1, -1), W3)

    part3 = _sc_scat32(g3, srcp, dstp, z32)
    return _tc_final(part3[0], part3[1], p3, dinv, b3.reshape(1, -1),
                     Watt, fcW, fcb.reshape(1, -1), sW, sb.reshape(1, -1))


# speedup vs baseline: 1.1279x; 1.1279x over previous
import functools

import jax
import jax.numpy as jnp
from jax import lax
from jax.experimental import pallas as pl
from jax.experimental.pallas import tpu as pltpu
from jax.experimental.pallas import tpu_sc as plsc

N = 10000
E = 320000
NPAD = 10016
NC, NS = 2, 16
NW = NC * NS
EPW = E // NW
CH = 112
NCH = -(-EPW // CH)
EPW_PAD = NCH * CH
RPT = NPAD // NS
DB = 16

_mesh = plsc.VectorSubcoreMesh(core_axis_name="c", subcore_axis_name="s",
                               num_cores=NC, num_subcores=NS)


def _make_sc_scatter(F):
    @functools.partial(
        pl.kernel,
        out_type=jax.ShapeDtypeStruct((NC, NPAD, F), jnp.float32),
        mesh=_mesh,
        compiler_params=pltpu.CompilerParams(use_tc_tiling_on_sc=False),
        scratch_types=[
            pltpu.VMEM((NCH, CH), jnp.int32),
            pltpu.VMEM((NCH, CH), jnp.int32),
            pltpu.VMEM((CH, F), jnp.float32),
            pltpu.VMEM((CH, F), jnp.float32),
            pltpu.VMEM_SHARED((NPAD, F), jnp.float32),
            pltpu.SemaphoreType.DMA,
            pltpu.SemaphoreType.DMA,
        ],
    )
    def k(g_hbm, src_hbm, dst_hbm, zeros_hbm, out_hbm,
          srciv, dstiv, gbuf0, gbuf1, acc, semg0, semg1):
        c = lax.axis_index("c")
        s = lax.axis_index("s")
        w = c * NS + s
        pltpu.sync_copy(src_hbm.at[w], srciv)
        pltpu.sync_copy(dst_hbm.at[w], dstiv)
        base = s * RPT
        pltpu.sync_copy(zeros_hbm, acc.at[pl.ds(base, RPT)])
        plsc.subcore_barrier()

        pltpu.async_copy(g_hbm.at[srciv.at[0]], gbuf0, semg0)
        pltpu.async_copy(g_hbm.at[srciv.at[1]], gbuf1, semg1)

        def body(jj, carry):
            j = jj * 2
            pltpu.make_async_copy(g_hbm.at[srciv.at[j]], gbuf0, semg0).wait()
            pltpu.sync_copy(gbuf0, acc.at[dstiv.at[j]], add=True)
            pltpu.async_copy(g_hbm.at[srciv.at[j + 2]], gbuf0, semg0)
            pltpu.make_async_copy(g_hbm.at[srciv.at[j + 1]], gbuf1, semg1).wait()
            pltpu.sync_copy(gbuf1, acc.at[dstiv.at[j + 1]], add=True)
            pltpu.async_copy(g_hbm.at[srciv.at[j + 3]], gbuf1, semg1)
            return carry

        lax.fori_loop(0, NCH // 2 - 1, body, 0)
        pltpu.make_async_copy(g_hbm.at[srciv.at[NCH - 2]], gbuf0, semg0).wait()
        pltpu.sync_copy(gbuf0, acc.at[dstiv.at[NCH - 2]], add=True)
        pltpu.make_async_copy(g_hbm.at[srciv.at[NCH - 1]], gbuf1, semg1).wait()
        pltpu.sync_copy(gbuf1, acc.at[dstiv.at[NCH - 1]], add=True)
        plsc.subcore_barrier()
        pltpu.sync_copy(acc.at[pl.ds(base, RPT)], out_hbm.at[c].at[pl.ds(base, RPT)])

    return k


@functools.partial(
    pl.kernel,
    out_type=jax.ShapeDtypeStruct((NC, NPAD, DB), jnp.float32),
    mesh=_mesh,
    compiler_params=pltpu.CompilerParams(use_tc_tiling_on_sc=False),
    scratch_types=[
        pltpu.VMEM((NCH, CH), jnp.int32),
        pltpu.VMEM((CH, DB), jnp.float32),
        pltpu.VMEM_SHARED((NPAD, DB), jnp.float32),
        pltpu.SemaphoreType.DMA,
    ],
)
def _sc_degree(dst_hbm, ones_hbm, zeros_hbm, out_hbm, dstiv, obuf, acc, sem):
    c = lax.axis_index("c")
    s = lax.axis_index("s")
    w = c * NS + s
    pltpu.sync_copy(dst_hbm.at[w], dstiv)
    pltpu.sync_copy(ones_hbm, obuf)
    base = s * RPT
    pltpu.sync_copy(zeros_hbm, acc.at[pl.ds(base, RPT)])
    plsc.subcore_barrier()

    def fire(j, carry):
        pltpu.async_copy(obuf, acc.at[dstiv.at[j]], sem, add=True)
        return carry

    def drain(j, carry):
        pltpu.make_async_copy(obuf, acc.at[dstiv.at[j]], sem).wait()
        return carry

    lax.fori_loop(0, NCH, fire, 0)
    lax.fori_loop(0, NCH, drain, 0)
    plsc.subcore_barrier()
    pltpu.sync_copy(acc.at[pl.ds(base, RPT)], out_hbm.at[c].at[pl.ds(base, RPT)])


_sc_scat128 = _make_sc_scatter(128)
_sc_scat64 = _make_sc_scatter(64)
_sc_scat32 = _make_sc_scatter(32)


def _tc_mm_body(x, w, p_o):
    p_o[...] = jnp.dot(x[...], w[...], preferred_element_type=jnp.float32,
                       precision=lax.Precision.HIGHEST)


def _tc_matmul(x, w):
    return pl.pallas_call(
        _tc_mm_body,
        out_shape=jax.ShapeDtypeStruct((NPAD, w.shape[1]), jnp.float32),
    )(x, w)


def _tc1_body(d0, d1, p, dinv_o, g_o):
    deg = d0[:, 0:1] + d1[:, 0:1] + 1.0
    dinv = lax.rsqrt(deg)
    dinv_o[...] = dinv
    g_o[...] = p[...] * dinv


def _tc_stage1(d0, d1, p):
    f = p.shape[1]
    return pl.pallas_call(
        _tc1_body,
        out_shape=(
            jax.ShapeDtypeStruct((NPAD, 1), jnp.float32),
            jax.ShapeDtypeStruct((NPAD, f), jnp.float32),
        ),
    )(d0, d1, p)


def _tc2_body(q0, q1, p_prev, dinv, b, w, p_o, g_o):
    dv = dinv[...]
    h = jnp.maximum((q0[...] + q1[...]) * dv + p_prev[...] * (dv * dv) + b[...], 0.0)
    p = jnp.dot(h, w[...], preferred_element_type=jnp.float32, precision=lax.Precision.HIGHEST)
    p_o[...] = p
    g_o[...] = p * dv


def _tc_stage2(q0, q1, p_prev, dinv, b, w):
    f = w.shape[1]
    return pl.pallas_call(
        _tc2_body,
        out_shape=(
            jax.ShapeDtypeStruct((NPAD, f), jnp.float32),
            jax.ShapeDtypeStruct((NPAD, f), jnp.float32),
        ),
    )(q0, q1, p_prev, dinv, b, w)


def _tc_final_body(q0, q1, p_prev, dinv, b, watt, fcw, fcb, sw, sb, out):
    dv = dinv[...]
    h3 = (q0[...] + q1[...]) * dv + p_prev[...] * (dv * dv) + b[...]
    mask = lax.broadcasted_iota(jnp.int32, (NPAD, 1), 0) < N
    h3m = jnp.where(mask, h3, 0.0)
    gw = jnp.dot(h3m, watt[...], preferred_element_type=jnp.float32, precision=lax.Precision.HIGHEST)
    gc = jnp.sum(gw, axis=0, keepdims=True) * (1.0 / N)
    tg = jnp.tanh(gc)
    ss = jax.nn.sigmoid(jnp.sum(h3m * tg, axis=1, keepdims=True))
    rep = jnp.sum(h3m * ss, axis=0, keepdims=True)
    s1 = jnp.maximum(jnp.dot(rep, fcw[...], preferred_element_type=jnp.float32, precision=lax.Precision.HIGHEST)
                     + fcb[...], 0.0)
    out[...] = jax.nn.sigmoid(
        jnp.dot(s1, sw[...], preferred_element_type=jnp.float32, precision=lax.Precision.HIGHEST) + sb[...])


def _tc_final(q0, q1, p_prev, dinv, b, watt, fcw, fcb, sw, sb):
    return pl.pallas_call(
        _tc_final_body,
        out_shape=jax.ShapeDtypeStruct((1, 1), jnp.float32),
    )(q0, q1, p_prev, dinv, b, watt, fcw, fcb, sw, sb)


def kernel(features_1, edge_index_1, W1, b1, W2, b2, W3, b3, Watt, fcW, fcb, sW, sb):
    ei = edge_index_1.astype(jnp.int32)
    src = ei[0].reshape(NW, EPW)
    dst = ei[1].reshape(NW, EPW)
    srcp = jnp.pad(src, ((0, 0), (0, EPW_PAD - EPW))).reshape(NW, NCH, CH)
    dstp = jnp.pad(dst, ((0, 0), (0, EPW_PAD - EPW)),
                   constant_values=N).reshape(NW, NCH, CH)
    xp = jnp.pad(features_1, ((0, NPAD - N), (0, 0)))

    z16 = jnp.zeros((RPT, DB), jnp.float32)
    z128 = jnp.zeros((RPT, 128), jnp.float32)
    z64 = jnp.zeros((RPT, 64), jnp.float32)
    z32 = jnp.zeros((RPT, 32), jnp.float32)
    ones16 = jnp.ones((CH, DB), jnp.float32)

    dpart = _sc_degree(dstp, ones16, z16)
    p1 = _tc_matmul(xp, W1)
    dinv, g1 = _tc_stage1(dpart[0], dpart[1], p1)

    part1 = _sc_scat128(g1, srcp, dstp, z128)
    p2, g2 = _tc_stage2(part1[0], part1[1], p1, dinv, b1.reshape(1, -1), W2)

    part2 = _sc_scat64(g2, srcp, dstp, z64)
    p3, g3 = _tc_stage2(part2[0], part2[1], p2, dinv, b2.reshape(1, -1), W3)

    part3 = _sc_scat32(g3, srcp, dstp, z32)
    return _tc_final(part3[0], part3[1], p3, dinv, b3.reshape(1, -1),
                     Watt, fcW, fcb.reshape(1, -1), sW, sb.reshape(1, -1))


# speedup vs baseline: 1.1489x; 1.0186x over previous
import functools

import jax
import jax.numpy as jnp
from jax import lax
from jax.experimental import pallas as pl
from jax.experimental.pallas import tpu as pltpu
from jax.experimental.pallas import tpu_sc as plsc

N = 10000
E = 320000
NPAD = 10016
NC, NS = 2, 16
NW = NC * NS
EPW = E // NW
CH = 120
NCH = -(-EPW // CH)
EPW_PAD = NCH * CH
RPT = NPAD // NS
DB = 16

_mesh = plsc.VectorSubcoreMesh(core_axis_name="c", subcore_axis_name="s",
                               num_cores=NC, num_subcores=NS)


def _make_sc_scatter(F):
    @functools.partial(
        pl.kernel,
        out_type=jax.ShapeDtypeStruct((NC, NPAD, F), jnp.float32),
        mesh=_mesh,
        compiler_params=pltpu.CompilerParams(use_tc_tiling_on_sc=False),
        scratch_types=[
            pltpu.VMEM((NCH, CH), jnp.int32),
            pltpu.VMEM((NCH, CH), jnp.int32),
            pltpu.VMEM((CH, F), jnp.float32),
            pltpu.VMEM((CH, F), jnp.float32),
            pltpu.VMEM_SHARED((NPAD, F), jnp.float32),
            pltpu.SemaphoreType.DMA,
            pltpu.SemaphoreType.DMA,
        ],
    )
    def k(g_hbm, src_hbm, dst_hbm, zeros_hbm, out_hbm,
          srciv, dstiv, gbuf0, gbuf1, acc, semg0, semg1):
        c = lax.axis_index("c")
        s = lax.axis_index("s")
        w = c * NS + s
        base = s * RPT
        pltpu.async_copy(src_hbm.at[w], srciv, semg0)
        pltpu.async_copy(dst_hbm.at[w], dstiv, semg1)
        pltpu.sync_copy(zeros_hbm, acc.at[pl.ds(base, RPT)])
        pltpu.make_async_copy(src_hbm.at[w], srciv, semg0).wait()
        pltpu.make_async_copy(dst_hbm.at[w], dstiv, semg1).wait()

        pltpu.async_copy(g_hbm.at[srciv.at[0]], gbuf0, semg0)
        pltpu.async_copy(g_hbm.at[srciv.at[1]], gbuf1, semg1)
        plsc.subcore_barrier()

        def body(jj, carry):
            j = jj * 2
            pltpu.make_async_copy(g_hbm.at[srciv.at[j]], gbuf0, semg0).wait()
            pltpu.sync_copy(gbuf0, acc.at[dstiv.at[j]], add=True)
            pltpu.async_copy(g_hbm.at[srciv.at[j + 2]], gbuf0, semg0)
            pltpu.make_async_copy(g_hbm.at[srciv.at[j + 1]], gbuf1, semg1).wait()
            pltpu.sync_copy(gbuf1, acc.at[dstiv.at[j + 1]], add=True)
            pltpu.async_copy(g_hbm.at[srciv.at[j + 3]], gbuf1, semg1)
            return carry

        lax.fori_loop(0, NCH // 2 - 1, body, 0)
        pltpu.make_async_copy(g_hbm.at[srciv.at[NCH - 2]], gbuf0, semg0).wait()
        pltpu.sync_copy(gbuf0, acc.at[dstiv.at[NCH - 2]], add=True)
        pltpu.make_async_copy(g_hbm.at[srciv.at[NCH - 1]], gbuf1, semg1).wait()
        pltpu.sync_copy(gbuf1, acc.at[dstiv.at[NCH - 1]], add=True)
        plsc.subcore_barrier()
        pltpu.sync_copy(acc.at[pl.ds(base, RPT)], out_hbm.at[c].at[pl.ds(base, RPT)])

    return k


@functools.partial(
    pl.kernel,
    out_type=jax.ShapeDtypeStruct((NC, NPAD, DB), jnp.float32),
    mesh=_mesh,
    compiler_params=pltpu.CompilerParams(use_tc_tiling_on_sc=False),
    scratch_types=[
        pltpu.VMEM((NCH, CH), jnp.int32),
        pltpu.VMEM((CH, DB), jnp.float32),
        pltpu.VMEM_SHARED((NPAD, DB), jnp.float32),
        pltpu.SemaphoreType.DMA,
    ],
)
def _sc_degree(dst_hbm, ones_hbm, zeros_hbm, out_hbm, dstiv, obuf, acc, sem):
    c = lax.axis_index("c")
    s = lax.axis_index("s")
    w = c * NS + s
    pltpu.sync_copy(dst_hbm.at[w], dstiv)
    pltpu.sync_copy(ones_hbm, obuf)
    base = s * RPT
    pltpu.sync_copy(zeros_hbm, acc.at[pl.ds(base, RPT)])
    plsc.subcore_barrier()

    def fire(j, carry):
        pltpu.async_copy(obuf, acc.at[dstiv.at[j]], sem, add=True)
        return carry

    def drain(j, carry):
        pltpu.make_async_copy(obuf, acc.at[dstiv.at[j]], sem).wait()
        return carry

    lax.fori_loop(0, NCH, fire, 0)
    lax.fori_loop(0, NCH, drain, 0)
    plsc.subcore_barrier()
    pltpu.sync_copy(acc.at[pl.ds(base, RPT)], out_hbm.at[c].at[pl.ds(base, RPT)])


_sc_scat128 = _make_sc_scatter(128)
_sc_scat64 = _make_sc_scatter(64)
_sc_scat32 = _make_sc_scatter(32)


def _tc_mm_body(x, w, p_o):
    p_o[...] = jnp.dot(x[...], w[...], preferred_element_type=jnp.float32,
                       precision=lax.Precision.HIGHEST)


def _tc_matmul(x, w):
    return pl.pallas_call(
        _tc_mm_body,
        out_shape=jax.ShapeDtypeStruct((NPAD, w.shape[1]), jnp.float32),
    )(x, w)


def _tc1_body(d0, d1, p, dinv_o, g_o):
    deg = d0[:, 0:1] + d1[:, 0:1] + 1.0
    dinv = lax.rsqrt(deg)
    dinv_o[...] = dinv
    g_o[...] = p[...] * dinv


def _tc_stage1(d0, d1, p):
    f = p.shape[1]
    return pl.pallas_call(
        _tc1_body,
        out_shape=(
            jax.ShapeDtypeStruct((NPAD, 1), jnp.float32),
            jax.ShapeDtypeStruct((NPAD, f), jnp.float32),
        ),
    )(d0, d1, p)


def _tc2_body(q0, q1, p_prev, dinv, b, w, p_o, g_o):
    dv = dinv[...]
    h = jnp.maximum((q0[...] + q1[...]) * dv + p_prev[...] * (dv * dv) + b[...], 0.0)
    p = jnp.dot(h, w[...], preferred_element_type=jnp.float32, precision=lax.Precision.HIGHEST)
    p_o[...] = p
    g_o[...] = p * dv


def _tc_stage2(q0, q1, p_prev, dinv, b, w):
    f = w.shape[1]
    return pl.pallas_call(
        _tc2_body,
        out_shape=(
            jax.ShapeDtypeStruct((NPAD, f), jnp.float32),
            jax.ShapeDtypeStruct((NPAD, f), jnp.float32),
        ),
    )(q0, q1, p_prev, dinv, b, w)


def _tc_final_body(q0, q1, p_prev, dinv, b, watt, fcw, fcb, sw, sb, out):
    dv = dinv[...]
    h3 = (q0[...] + q1[...]) * dv + p_prev[...] * (dv * dv) + b[...]
    mask = lax.broadcasted_iota(jnp.int32, (NPAD, 1), 0) < N
    h3m = jnp.where(mask, h3, 0.0)
    gw = jnp.dot(h3m, watt[...], preferred_element_type=jnp.float32, precision=lax.Precision.HIGHEST)
    gc = jnp.sum(gw, axis=0, keepdims=True) * (1.0 / N)
    tg = jnp.tanh(gc)
    ss = jax.nn.sigmoid(jnp.sum(h3m * tg, axis=1, keepdims=True))
    rep = jnp.sum(h3m * ss, axis=0, keepdims=True)
    s1 = jnp.maximum(jnp.dot(rep, fcw[...], preferred_element_type=jnp.float32, precision=lax.Precision.HIGHEST)
                     + fcb[...], 0.0)
    out[...] = jax.nn.sigmoid(
        jnp.dot(s1, sw[...], preferred_element_type=jnp.float32, precision=lax.Precision.HIGHEST) + sb[...])


def _tc_final(q0, q1, p_prev, dinv, b, watt, fcw, fcb, sw, sb):
    return pl.pallas_call(
        _tc_final_body,
        out_shape=jax.ShapeDtypeStruct((1, 1), jnp.float32),
    )(q0, q1, p_prev, dinv, b, watt, fcw, fcb, sw, sb)


def kernel(features_1, edge_index_1, W1, b1, W2, b2, W3, b3, Watt, fcW, fcb, sW, sb):
    ei = edge_index_1.astype(jnp.int32)
    src = ei[0].reshape(NW, EPW)
    dst = ei[1].reshape(NW, EPW)
    srcp = jnp.pad(src, ((0, 0), (0, EPW_PAD - EPW))).reshape(NW, NCH, CH)
    dstp = jnp.pad(dst, ((0, 0), (0, EPW_PAD - EPW)),
                   constant_values=N).reshape(NW, NCH, CH)
    xp = jnp.pad(features_1, ((0, NPAD - N), (0, 0)))

    z16 = jnp.zeros((RPT, DB), jnp.float32)
    z128 = jnp.zeros((RPT, 128), jnp.float32)
    z64 = jnp.zeros((RPT, 64), jnp.float32)
    z32 = jnp.zeros((RPT, 32), jnp.float32)
    ones16 = jnp.ones((CH, DB), jnp.float32)

    dpart = _sc_degree(dstp, ones16, z16)
    p1 = _tc_matmul(xp, W1)
    dinv, g1 = _tc_stage1(dpart[0], dpart[1], p1)

    part1 = _sc_scat128(g1, srcp, dstp, z128)
    p2, g2 = _tc_stage2(part1[0], part1[1], p1, dinv, b1.reshape(1, -1), W2)

    part2 = _sc_scat64(g2, srcp, dstp, z64)
    p3, g3 = _tc_stage2(part2[0], part2[1], p2, dinv, b2.reshape(1, -1), W3)

    part3 = _sc_scat32(g3, srcp, dstp, z32)
    return _tc_final(part3[0], part3[1], p3, dinv, b3.reshape(1, -1),
                     Watt, fcW, fcb.reshape(1, -1), sW, sb.reshape(1, -1))
